# baseline (speedup 1.0000x reference)
import numpy as np
import jax
import jax.numpy as jnp
from jax.experimental import pallas as pl
from jax.experimental.pallas import tpu as pltpu

_H, _D = 16, 64
_C, _ITERS, _BITS = 100, 10, 32
_CP = 128
_RB = 256


def _qkv_kernel(x_ref, wq_ref, bq_ref, wk_ref, bk_ref, wv_ref, bv_ref,
                q_ref, k_ref, v_ref):
    xb = x_ref[...]
    q_ref[...] = jnp.dot(xb, wq_ref[...]) + bq_ref[...]
    k_ref[...] = jnp.dot(xb, wk_ref[...]) + bk_ref[...]
    v_ref[...] = jnp.dot(xb, wv_ref[...]) + bv_ref[...]


def _out_kernel(o_ref, wo_ref, bo_ref, out_ref):
    out_ref[...] = jnp.dot(o_ref[...], wo_ref[...]) + bo_ref[...]


def _attn_kernel(q_ref, k_ref, v_ref, hp_ref, init_ref, og_ref):
    S = q_ref.shape[2]
    temp = 1.0 / np.sqrt(_D)

    q = q_ref[0, 0]
    k = k_ref[0, 0]
    v = v_ref[0, 0]

    proj = jnp.dot(q, hp_ref[...])
    codes = jnp.where(proj > 0, 1.0, -1.0).astype(jnp.float32)

    c_iota = jax.lax.broadcasted_iota(jnp.int32, (S, _CP), 1)
    pad_mask = jnp.where(c_iota >= _C, 1e30, 0.0)
    s_iota = jax.lax.broadcasted_iota(jnp.int32, (_CP, S), 1)
    sel = (s_iota == init_ref[...]).astype(jnp.float32)
    cent0 = jax.lax.dot_general(sel, codes, (((1,), (0,)), ((), ())))

    ones_s = jnp.ones((S, 1), jnp.float32)
    code_sq = jnp.sum(codes * codes, axis=-1, keepdims=True)

    def _one_hot(cent):
        m = jax.lax.dot_general(codes, cent, (((1,), (1,)), ((), ())))
        centsq = jnp.sum(cent * cent, axis=-1, keepdims=True).T
        d = code_sq - 2.0 * m + centsq + pad_mask
        dmin = jnp.min(d, axis=-1, keepdims=True)
        am = jnp.where(d == dmin, c_iota, _CP)
        assign = jnp.min(am, axis=-1, keepdims=True)
        return (c_iota == assign).astype(jnp.float32)

    def _body(_, cent):
        oh = _one_hot(cent)
        cnt = jnp.maximum(
            jax.lax.dot_general(oh, ones_s, (((0,), (0,)), ((), ()))), 1.0)
        return jax.lax.dot_general(oh, codes, (((0,), (0,)), ((), ()))) / cnt

    cent = jax.lax.fori_loop(0, _ITERS - 1, _body, cent0)
    oh = _one_hot(cent)

    cnt = jnp.maximum(
        jax.lax.dot_general(oh, ones_s, (((0,), (0,)), ((), ()))), 1.0)
    qc = jax.lax.dot_general(oh, q, (((0,), (0,)), ((), ()))) / cnt
    logits = temp * jax.lax.dot_general(qc, k, (((1,), (1,)), ((), ())))
    mx = jnp.max(logits, axis=-1, keepdims=True)
    e = jnp.exp(logits - mx)
    a = e / jnp.sum(e, axis=-1, keepdims=True)
    oc = jax.lax.dot_general(a, v, (((1,), (0,)), ((), ())))
    og_ref[0, 0] = jnp.dot(oh, oc)


def kernel(x, attention_mask, Wq, bq, Wk, bk, Wv, bv, Wo, bo, hash_planes):
    del attention_mask
    Bx, Sx, Ex = x.shape
    HD = _H * _D
    M = Bx * Sx

    init_idx = np.full((_CP, 1), -1, np.int32)
    init_idx[:_C, 0] = np.linspace(0, Sx - 1, _C).astype(np.int32)
    init_idx = jnp.asarray(init_idx)

    q_flat, k_flat, v_flat = pl.pallas_call(
        _qkv_kernel,
        grid=(M // _RB,),
        in_specs=[
            pl.BlockSpec((_RB, Ex), lambda i: (i, 0)),
            pl.BlockSpec((Ex, HD), lambda i: (0, 0)),
            pl.BlockSpec((1, HD), lambda i: (0, 0)),
            pl.BlockSpec((Ex, HD), lambda i: (0, 0)),
            pl.BlockSpec((1, HD), lambda i: (0, 0)),
            pl.BlockSpec((Ex, HD), lambda i: (0, 0)),
            pl.BlockSpec((1, HD), lambda i: (0, 0)),
        ],
        out_specs=[
            pl.BlockSpec((_RB, HD), lambda i: (i, 0)),
            pl.BlockSpec((_RB, HD), lambda i: (i, 0)),
            pl.BlockSpec((_RB, HD), lambda i: (i, 0)),
        ],
        out_shape=[jax.ShapeDtypeStruct((M, HD), jnp.float32)] * 3,
        compiler_params=pltpu.CompilerParams(
            dimension_semantics=("arbitrary",)),
        interpret=False,
    )(x.reshape(M, Ex), Wq, bq.reshape(1, HD), Wk, bk.reshape(1, HD),
      Wv, bv.reshape(1, HD))

    to4 = lambda t: t.reshape(Bx, Sx, _H, _D).transpose(0, 2, 1, 3)
    q4, k4, v4 = to4(q_flat), to4(k_flat), to4(v_flat)

    og4 = pl.pallas_call(
        _attn_kernel,
        grid=(Bx, _H),
        in_specs=[
            pl.BlockSpec((1, 1, Sx, _D), lambda b, h: (b, h, 0, 0)),
            pl.BlockSpec((1, 1, Sx, _D), lambda b, h: (b, h, 0, 0)),
            pl.BlockSpec((1, 1, Sx, _D), lambda b, h: (b, h, 0, 0)),
            pl.BlockSpec((_D, _BITS), lambda b, h: (0, 0)),
            pl.BlockSpec((_CP, 1), lambda b, h: (0, 0)),
        ],
        out_specs=pl.BlockSpec((1, 1, Sx, _D), lambda b, h: (b, h, 0, 0)),
        out_shape=jax.ShapeDtypeStruct((Bx, _H, Sx, _D), jnp.float32),
        compiler_params=pltpu.CompilerParams(
            dimension_semantics=("arbitrary", "arbitrary")),
        interpret=False,
    )(q4, k4, v4, hash_planes, init_idx)

    o_flat = og4.transpose(0, 2, 1, 3).reshape(M, HD)

    out = pl.pallas_call(
        _out_kernel,
        grid=(M // _RB,),
        in_specs=[
            pl.BlockSpec((_RB, HD), lambda i: (i, 0)),
            pl.BlockSpec((HD, Ex), lambda i: (0, 0)),
            pl.BlockSpec((1, Ex), lambda i: (0, 0)),
        ],
        out_specs=pl.BlockSpec((_RB, Ex), lambda i: (i, 0)),
        out_shape=jax.ShapeDtypeStruct((M, Ex), jnp.float32),
        compiler_params=pltpu.CompilerParams(
            dimension_semantics=("arbitrary",)),
        interpret=False,
    )(o_flat, Wo, bo.reshape(1, Ex))
    return out.reshape(Bx, Sx, Ex)


# speedup vs baseline: 1.0670x; 1.0670x over previous
import numpy as np
import jax
import jax.numpy as jnp
from jax.experimental import pallas as pl
from jax.experimental.pallas import tpu as pltpu

_H, _D = 16, 64
_C, _ITERS, _BITS = 100, 10, 32
_CP = 128
_RB = 256


def _qkv_kernel(x_ref, wq_ref, bq_ref, wk_ref, bk_ref, wv_ref, bv_ref,
                q_ref, k_ref, v_ref):
    xb = x_ref[...]
    q_ref[...] = jnp.dot(xb, wq_ref[...]) + bq_ref[...]
    k_ref[...] = jnp.dot(xb, wk_ref[...]) + bk_ref[...]
    v_ref[...] = jnp.dot(xb, wv_ref[...]) + bv_ref[...]


def _out_kernel(o_ref, wo_ref, bo_ref, out_ref):
    out_ref[...] = jnp.dot(o_ref[...], wo_ref[...]) + bo_ref[...]


def _attn_kernel(q_ref, k_ref, v_ref, hp_ref, init_ref, og_ref):
    S = q_ref.shape[2]
    temp = 1.0 / np.sqrt(_D)

    q = q_ref[0, 0]
    k = k_ref[0, 0]
    v = v_ref[0, 0]

    proj = jnp.dot(q, hp_ref[...])
    codes = jnp.where(proj > 0, 1.0, -1.0).astype(jnp.float32)

    c_iota = jax.lax.broadcasted_iota(jnp.int32, (S, _CP), 1)
    pad_mask = jnp.where(c_iota >= _C, 1e30, 0.0)
    s_iota = jax.lax.broadcasted_iota(jnp.int32, (_CP, S), 1)
    sel = (s_iota == init_ref[...]).astype(jnp.float32)
    cent0 = jax.lax.dot_general(sel, codes, (((1,), (0,)), ((), ())))

    ones_s = jnp.ones((S, 1), jnp.float32)
    code_sq = jnp.sum(codes * codes, axis=-1, keepdims=True)

    def _one_hot(cent):
        m = jax.lax.dot_general(codes, cent, (((1,), (1,)), ((), ())))
        centsq = jnp.sum(cent * cent, axis=-1, keepdims=True).T
        d = code_sq - 2.0 * m + centsq + pad_mask
        dmin = jnp.min(d, axis=-1, keepdims=True)
        am = jnp.where(d == dmin, c_iota, _CP)
        assign = jnp.min(am, axis=-1, keepdims=True)
        return (c_iota == assign).astype(jnp.float32)

    def _body(_, cent):
        oh = _one_hot(cent)
        cnt = jnp.maximum(
            jax.lax.dot_general(oh, ones_s, (((0,), (0,)), ((), ()))), 1.0)
        return jax.lax.dot_general(oh, codes, (((0,), (0,)), ((), ()))) / cnt

    cent = cent0
    for _ in range(_ITERS - 1):
        cent = _body(None, cent)
    oh = _one_hot(cent)

    cnt = jnp.maximum(
        jax.lax.dot_general(oh, ones_s, (((0,), (0,)), ((), ()))), 1.0)
    qc = jax.lax.dot_general(oh, q, (((0,), (0,)), ((), ()))) / cnt
    logits = temp * jax.lax.dot_general(qc, k, (((1,), (1,)), ((), ())))
    mx = jnp.max(logits, axis=-1, keepdims=True)
    e = jnp.exp(logits - mx)
    a = e / jnp.sum(e, axis=-1, keepdims=True)
    oc = jax.lax.dot_general(a, v, (((1,), (0,)), ((), ())))
    og_ref[0, 0] = jnp.dot(oh, oc)


def kernel(x, attention_mask, Wq, bq, Wk, bk, Wv, bv, Wo, bo, hash_planes):
    del attention_mask
    Bx, Sx, Ex = x.shape
    HD = _H * _D
    M = Bx * Sx

    init_idx = np.full((_CP, 1), -1, np.int32)
    init_idx[:_C, 0] = np.linspace(0, Sx - 1, _C).astype(np.int32)
    init_idx = jnp.asarray(init_idx)

    q_flat, k_flat, v_flat = pl.pallas_call(
        _qkv_kernel,
        grid=(M // _RB,),
        in_specs=[
            pl.BlockSpec((_RB, Ex), lambda i: (i, 0)),
            pl.BlockSpec((Ex, HD), lambda i: (0, 0)),
            pl.BlockSpec((1, HD), lambda i: (0, 0)),
            pl.BlockSpec((Ex, HD), lambda i: (0, 0)),
            pl.BlockSpec((1, HD), lambda i: (0, 0)),
            pl.BlockSpec((Ex, HD), lambda i: (0, 0)),
            pl.BlockSpec((1, HD), lambda i: (0, 0)),
        ],
        out_specs=[
            pl.BlockSpec((_RB, HD), lambda i: (i, 0)),
            pl.BlockSpec((_RB, HD), lambda i: (i, 0)),
            pl.BlockSpec((_RB, HD), lambda i: (i, 0)),
        ],
        out_shape=[jax.ShapeDtypeStruct((M, HD), jnp.float32)] * 3,
        compiler_params=pltpu.CompilerParams(
            dimension_semantics=("arbitrary",)),
        interpret=False,
    )(x.reshape(M, Ex), Wq, bq.reshape(1, HD), Wk, bk.reshape(1, HD),
      Wv, bv.reshape(1, HD))

    to4 = lambda t: t.reshape(Bx, Sx, _H, _D).transpose(0, 2, 1, 3)
    q4, k4, v4 = to4(q_flat), to4(k_flat), to4(v_flat)

    og4 = pl.pallas_call(
        _attn_kernel,
        grid=(Bx, _H),
        in_specs=[
            pl.BlockSpec((1, 1, Sx, _D), lambda b, h: (b, h, 0, 0)),
            pl.BlockSpec((1, 1, Sx, _D), lambda b, h: (b, h, 0, 0)),
            pl.BlockSpec((1, 1, Sx, _D), lambda b, h: (b, h, 0, 0)),
            pl.BlockSpec((_D, _BITS), lambda b, h: (0, 0)),
            pl.BlockSpec((_CP, 1), lambda b, h: (0, 0)),
        ],
        out_specs=pl.BlockSpec((1, 1, Sx, _D), lambda b, h: (b, h, 0, 0)),
        out_shape=jax.ShapeDtypeStruct((Bx, _H, Sx, _D), jnp.float32),
        compiler_params=pltpu.CompilerParams(
            dimension_semantics=("arbitrary", "arbitrary")),
        interpret=False,
    )(q4, k4, v4, hash_planes, init_idx)

    o_flat = og4.transpose(0, 2, 1, 3).reshape(M, HD)

    out = pl.pallas_call(
        _out_kernel,
        grid=(M // _RB,),
        in_specs=[
            pl.BlockSpec((_RB, HD), lambda i: (i, 0)),
            pl.BlockSpec((HD, Ex), lambda i: (0, 0)),
            pl.BlockSpec((1, Ex), lambda i: (0, 0)),
        ],
        out_specs=pl.BlockSpec((_RB, Ex), lambda i: (i, 0)),
        out_shape=jax.ShapeDtypeStruct((M, Ex), jnp.float32),
        compiler_params=pltpu.CompilerParams(
            dimension_semantics=("arbitrary",)),
        interpret=False,
    )(o_flat, Wo, bo.reshape(1, Ex))
    return out.reshape(Bx, Sx, Ex)
